# baseline (device time: 43062 ns/iter reference)
import jax
import jax.numpy as jnp
from jax import lax
from jax.experimental import pallas as pl
from jax.experimental.pallas import tpu as pltpu

N_DEV = 8
B, SQ, SKV, HQ, DH = 2, 128, 128, 32, 64
H_LOC = HQ // N_DEV
DMODEL = 512
ROWS = B * SQ


def kernel(x, Wq, K_ext, V_ext, Wo):
    my_i = lax.axis_index("i")
    k_loc = lax.dynamic_slice(K_ext, (0, 0, my_i * H_LOC, 0), (B, SKV, H_LOC, DH))
    v_loc = lax.dynamic_slice(V_ext, (0, 0, my_i * H_LOC, 0), (B, SKV, H_LOC, DH))

    def body(x_ref, wq_ref, k_ref, v_ref, wo_ref, out_ref,
             mine_ref, comm_ref, send_sems, recv_sems):
        me = lax.axis_index("i")

        barrier_sem = pltpu.get_barrier_semaphore()
        for d in range(1, N_DEV):
            tgt = lax.rem(me + d, N_DEV)
            pl.semaphore_signal(
                barrier_sem, inc=1,
                device_id=(tgt,), device_id_type=pl.DeviceIdType.MESH,
            )
        pl.semaphore_wait(barrier_sem, N_DEV - 1)

        x2d = x_ref[...].reshape(ROWS, DMODEL)
        q = jnp.dot(x2d, wq_ref[...], preferred_element_type=jnp.float32)

        qb = lax.broadcasted_iota(jnp.int32, (SQ, SKV), 0) // 64
        kb = lax.broadcasted_iota(jnp.int32, (SQ, SKV), 1) // 64
        mask = (qb == kb) | ((kb % 4) == (qb % 4))

        ctx_rows = []
        for b in range(B):
            ctx_cols = []
            for h in range(H_LOC):
                q_bh = q[b * SQ:(b + 1) * SQ, h * DH:(h + 1) * DH]
                k_bh = k_ref[b, :, h, :]
                v_bh = v_ref[b, :, h, :]
                s = lax.dot_general(
                    q_bh, k_bh, (((1,), (1,)), ((), ())),
                    preferred_element_type=jnp.float32,
                ) * 0.125
                s = jnp.where(mask, s, -1e9)
                s = s - jnp.max(s, axis=-1, keepdims=True)
                w = jnp.exp(s)
                w = w / jnp.sum(w, axis=-1, keepdims=True)
                ctx_cols.append(
                    jnp.dot(w, v_bh, preferred_element_type=jnp.float32))
            ctx_rows.append(jnp.concatenate(ctx_cols, axis=1))
        ctx = jnp.concatenate(ctx_rows, axis=0)
        partial = jnp.dot(ctx, wo_ref[...],
                          preferred_element_type=jnp.float32)
        mine_ref[...] = partial

        rdmas = []
        for d in range(1, N_DEV):
            tgt = lax.rem(me + d, N_DEV)
            rdma = pltpu.make_async_remote_copy(
                src_ref=mine_ref,
                dst_ref=comm_ref.at[d - 1],
                send_sem=send_sems.at[d - 1],
                recv_sem=recv_sems.at[d - 1],
                device_id=(tgt,),
                device_id_type=pl.DeviceIdType.MESH,
            )
            rdma.start()
            rdmas.append(rdma)

        acc = partial
        for d in range(1, N_DEV):
            rdmas[d - 1].wait_recv()
            acc = acc + comm_ref[d - 1]
        for d in range(1, N_DEV):
            rdmas[d - 1].wait_send()

        out_ref[...] = acc.reshape(B, SQ, DMODEL)

    return pl.pallas_call(
        body,
        out_shape=jax.ShapeDtypeStruct((B, SQ, DMODEL), jnp.float32),
        in_specs=[pl.BlockSpec(memory_space=pltpu.VMEM)] * 5,
        out_specs=pl.BlockSpec(memory_space=pltpu.VMEM),
        scratch_shapes=[
            pltpu.VMEM((ROWS, DMODEL), jnp.float32),
            pltpu.VMEM((N_DEV - 1, ROWS, DMODEL), jnp.float32),
            pltpu.SemaphoreType.DMA((N_DEV - 1,)),
            pltpu.SemaphoreType.DMA((N_DEV - 1,)),
        ],
        compiler_params=pltpu.CompilerParams(collective_id=0),
    )(x, Wq, k_loc, v_loc, Wo)


# device time: 5559 ns/iter; 7.7464x vs baseline; 7.7464x over previous
import jax
import jax.numpy as jnp
from jax import lax
from jax.experimental import pallas as pl
from jax.experimental.pallas import tpu as pltpu

N_DEV = 8
B, SQ, SKV, HQ, DH = 2, 128, 128, 32, 64
H_LOC = HQ // N_DEV
DMODEL = 512
ROWS = B * SQ
CH = ROWS // N_DEV


def kernel(x, Wq, K_ext, V_ext, Wo):
    my_i = lax.axis_index("i")
    k_loc = lax.dynamic_slice(K_ext, (0, 0, my_i * H_LOC, 0), (B, SKV, H_LOC, DH))
    v_loc = lax.dynamic_slice(V_ext, (0, 0, my_i * H_LOC, 0), (B, SKV, H_LOC, DH))

    def body(x_ref, wq_ref, k_ref, v_ref, wo_ref, out_ref,
             mine_ref, rs_ref, red_ref, ag_ref, o2d_ref,
             rs_send, rs_recv, ag_send, ag_recv):
        me = lax.axis_index("i")

        barrier_sem = pltpu.get_barrier_semaphore()
        for d in range(1, N_DEV):
            tgt = lax.rem(me + d, N_DEV)
            pl.semaphore_signal(
                barrier_sem, inc=1,
                device_id=(tgt,), device_id_type=pl.DeviceIdType.MESH,
            )
        pl.semaphore_wait(barrier_sem, N_DEV - 1)

        x2d = x_ref[...].reshape(ROWS, DMODEL)
        q = jnp.dot(x2d, wq_ref[...], preferred_element_type=jnp.float32)

        qb = lax.broadcasted_iota(jnp.int32, (SQ, SKV), 0) // 64
        kb = lax.broadcasted_iota(jnp.int32, (SQ, SKV), 1) // 64
        mask = (qb == kb) | ((kb % 4) == (qb % 4))

        ctx_rows = []
        for b in range(B):
            ctx_cols = []
            for h in range(H_LOC):
                q_bh = q[b * SQ:(b + 1) * SQ, h * DH:(h + 1) * DH]
                k_bh = k_ref[b, :, h, :]
                v_bh = v_ref[b, :, h, :]
                s = lax.dot_general(
                    q_bh, k_bh, (((1,), (1,)), ((), ())),
                    preferred_element_type=jnp.float32,
                ) * 0.125
                s = jnp.where(mask, s, -1e9)
                s = s - jnp.max(s, axis=-1, keepdims=True)
                w = jnp.exp(s)
                w = w / jnp.sum(w, axis=-1, keepdims=True)
                ctx_cols.append(
                    jnp.dot(w, v_bh, preferred_element_type=jnp.float32))
            ctx_rows.append(jnp.concatenate(ctx_cols, axis=1))
        ctx = jnp.concatenate(ctx_rows, axis=0)
        partial = jnp.dot(ctx, wo_ref[...],
                          preferred_element_type=jnp.float32)
        mine_ref[...] = partial

        rs = []
        for d in range(1, N_DEV):
            tgt = lax.rem(me + d, N_DEV)
            rdma = pltpu.make_async_remote_copy(
                src_ref=mine_ref.at[pl.ds(tgt * CH, CH), :],
                dst_ref=rs_ref.at[d - 1],
                send_sem=rs_send.at[d - 1],
                recv_sem=rs_recv.at[d - 1],
                device_id=(tgt,),
                device_id_type=pl.DeviceIdType.MESH,
            )
            rdma.start()
            rs.append(rdma)

        red = mine_ref[pl.ds(me * CH, CH), :]
        for d in range(1, N_DEV):
            rs[d - 1].wait_recv()
            red = red + rs_ref[d - 1]
        red_ref[...] = red

        ag = []
        for d in range(1, N_DEV):
            tgt = lax.rem(me + d, N_DEV)
            rdma = pltpu.make_async_remote_copy(
                src_ref=red_ref,
                dst_ref=ag_ref.at[d - 1],
                send_sem=ag_send.at[d - 1],
                recv_sem=ag_recv.at[d - 1],
                device_id=(tgt,),
                device_id_type=pl.DeviceIdType.MESH,
            )
            rdma.start()
            ag.append(rdma)

        o2d_ref[pl.ds(me * CH, CH), :] = red
        for d in range(1, N_DEV):
            ag[d - 1].wait_recv()
            src = lax.rem(me - d + N_DEV, N_DEV)
            o2d_ref[pl.ds(src * CH, CH), :] = ag_ref[d - 1]

        for d in range(1, N_DEV):
            rs[d - 1].wait_send()
            ag[d - 1].wait_send()

        out_ref[...] = o2d_ref[...].reshape(B, SQ, DMODEL)

    return pl.pallas_call(
        body,
        out_shape=jax.ShapeDtypeStruct((B, SQ, DMODEL), jnp.float32),
        in_specs=[pl.BlockSpec(memory_space=pltpu.VMEM)] * 5,
        out_specs=pl.BlockSpec(memory_space=pltpu.VMEM),
        scratch_shapes=[
            pltpu.VMEM((ROWS, DMODEL), jnp.float32),
            pltpu.VMEM((N_DEV - 1, CH, DMODEL), jnp.float32),
            pltpu.VMEM((CH, DMODEL), jnp.float32),
            pltpu.VMEM((N_DEV - 1, CH, DMODEL), jnp.float32),
            pltpu.VMEM((ROWS, DMODEL), jnp.float32),
            pltpu.SemaphoreType.DMA((N_DEV - 1,)),
            pltpu.SemaphoreType.DMA((N_DEV - 1,)),
            pltpu.SemaphoreType.DMA((N_DEV - 1,)),
            pltpu.SemaphoreType.DMA((N_DEV - 1,)),
        ],
        compiler_params=pltpu.CompilerParams(collective_id=0),
    )(x, Wq, k_loc, v_loc, Wo)
